# Initial kernel scaffold; baseline (speedup 1.0000x reference)
import functools

import jax
import jax.numpy as jnp
from jax import lax
from jax.experimental import pallas as pl
from jax.experimental.pallas import tpu as pltpu
from jax.experimental.pallas import tpu_sc as plsc

N_NODES = 10000
N_EDGES = 320000
D_IN = 128
D_OUT = 64
N_GRAPHS = 2500

NC = 2
NS = 16
NW = NC * NS
E_PER_TILE = N_EDGES // NW
G = 80
NGRP = E_PER_TILE // G
N_PAD = 10240
ROWS_PER_SUB = N_PAD // NS


def _dense_body(x_ref, anc_ref, w_ref, b_ref, a_ref, h_ref, anc_out_ref):
    w = w_ref[...]
    h_ref[...] = jnp.dot(x_ref[...], w, preferred_element_type=jnp.float32)
    v = jnp.dot(anc_ref[...], w, preferred_element_type=jnp.float32) + b_ref[...]
    a = a_ref[0, 0]
    anc_out_ref[...] = jnp.maximum(v, 0.0) + a * jnp.minimum(v, 0.0)


_dense_call = pl.pallas_call(
    _dense_body,
    out_shape=(
        jax.ShapeDtypeStruct((N_NODES, D_OUT), jnp.float32),
        jax.ShapeDtypeStruct((N_GRAPHS, D_OUT), jnp.float32),
    ),
)


_vector_mesh = plsc.VectorSubcoreMesh(core_axis_name="c", subcore_axis_name="s")


@functools.partial(
    pl.kernel,
    out_type=jax.ShapeDtypeStruct((NC, N_PAD, D_OUT), jnp.float32),
    mesh=_vector_mesh,
    scratch_types=[
        pltpu.VMEM((G,), jnp.int32),
        pltpu.VMEM((G,), jnp.int32),
        pltpu.VMEM((G,), jnp.float32),
        pltpu.VMEM((G, D_OUT), jnp.float32),
        pltpu.VMEM_SHARED((N_PAD, D_OUT), jnp.float32),
        pltpu.SemaphoreType.DMA,
    ],
)
def _edge_kernel(h_hbm, src_hbm, dst_hbm, ew_hbm, out_hbm,
                 src_v, dst_v, ew_v, rows_v, accum, sem):
    c = lax.axis_index("c")
    s = lax.axis_index("s")

    zero16 = jnp.zeros((16,), jnp.float32)

    @pl.loop(0, G)
    def _(i):
        for j in range(D_OUT // 16):
            rows_v[i, pl.ds(j * 16, 16)] = zero16

    @pl.loop(0, ROWS_PER_SUB // G)
    def _(k):
        pltpu.sync_copy(rows_v, accum.at[pl.ds(s * ROWS_PER_SUB + k * G, G)])

    plsc.subcore_barrier()

    base = (c * NS + s) * E_PER_TILE

    @pl.loop(0, NGRP)
    def _(g):
        off = base + g * G
        pltpu.sync_copy(src_hbm.at[pl.ds(off, G)], src_v)
        pltpu.sync_copy(ew_hbm.at[pl.ds(off, G)], ew_v)
        pltpu.sync_copy(dst_hbm.at[pl.ds(off, G)], dst_v)
        pltpu.async_copy(h_hbm.at[src_v], rows_v, sem).wait()

        @pl.loop(0, G)
        def _(i):
            w = plsc.load_gather(ew_v, [jnp.full((16,), i, jnp.int32)])
            for j in range(D_OUT // 16):
                sl = (i, pl.ds(j * 16, 16))
                rows_v[sl] = rows_v[sl] * w

        pltpu.sync_copy(rows_v, accum.at[dst_v], add=True)

    plsc.subcore_barrier()
    pltpu.sync_copy(accum.at[pl.ds(s * ROWS_PER_SUB, ROWS_PER_SUB)],
                    out_hbm.at[c].at[pl.ds(s * ROWS_PER_SUB, ROWS_PER_SUB)])


_POOL_BS = 500
_POOL_NB = N_NODES // _POOL_BS


def _post_body(part_ref, gid_ref, b_ref, a_ref, anc_ref, wb_ref, bb_ref,
               h2_ref, pool_ref, logits_ref):
    agg = part_ref[0, :N_NODES, :] + part_ref[1, :N_NODES, :]
    v = agg + b_ref[...]
    a = a_ref[0, 0]
    h2 = jnp.maximum(v, 0.0) + a * jnp.minimum(v, 0.0)
    h2_ref[...] = h2

    gidb = gid_ref[...].reshape(_POOL_NB, _POOL_BS, 1)
    h2b = h2.reshape(_POOL_NB, _POOL_BS, D_OUT)
    giota = lax.broadcasted_iota(jnp.int32, (1, N_GRAPHS), 1)
    ones_col = jnp.ones((_POOL_BS, 1), jnp.float32)

    def body(i, acc):
        ids = lax.dynamic_index_in_dim(gidb, i, 0, keepdims=False)
        oh = (ids == giota).astype(jnp.float32)
        hb = lax.dynamic_index_in_dim(h2b, i, 0, keepdims=False)
        m = jnp.concatenate([hb, ones_col], axis=1)
        return acc + lax.dot_general(oh, m, (((0,), (0,)), ((), ())),
                                     preferred_element_type=jnp.float32)

    acc = lax.fori_loop(0, _POOL_NB, body,
                        jnp.zeros((N_GRAPHS, D_OUT + 1), jnp.float32))
    pool = acc[:, :D_OUT] / jnp.maximum(acc[:, D_OUT:], 1.0)
    pool_ref[...] = pool
    t = jnp.dot(anc_ref[...], wb_ref[...], preferred_element_type=jnp.float32)
    logits_ref[...] = jnp.sum(t * pool, axis=1, keepdims=True) + bb_ref[...]


_post_call = pl.pallas_call(
    _post_body,
    out_shape=(
        jax.ShapeDtypeStruct((N_NODES, D_OUT), jnp.float32),
        jax.ShapeDtypeStruct((N_GRAPHS, D_OUT), jnp.float32),
        jax.ShapeDtypeStruct((N_GRAPHS, 1), jnp.float32),
    ),
)


def kernel(x, edge_index, edge_weight, graph_ids, anchor_embs, W, b, prelu_a, Wb, bb):
    src = edge_index[0].astype(jnp.int32)
    dst = edge_index[1].astype(jnp.int32)
    gid = graph_ids.astype(jnp.int32).reshape(N_NODES, 1)
    b2 = b.reshape(1, D_OUT)
    a2 = prelu_a.reshape(1, 1)
    bb2 = bb.reshape(1, 1)

    h, anchor_out = _dense_call(x, anchor_embs, W, b2, a2)
    part = _edge_kernel(h, src, dst, edge_weight)
    h2, pool, logits = _post_call(part, gid, b2, a2, anchor_out, Wb[0], bb2)
    return (h2, pool, anchor_out, logits)


# speedup vs baseline: 3.9911x; 3.9911x over previous
import dataclasses
import functools

import jax
import jax.numpy as jnp
from jax import lax
from jax.experimental import pallas as pl
from jax.experimental.pallas import tpu as pltpu
from jax.experimental.pallas import tpu_sc as plsc

N_NODES = 10000
N_EDGES = 320000
D_IN = 128
D_OUT = 64
N_GRAPHS = 2500

NC = 2
NS = 16
NW = NC * NS
E_PER_TILE = N_EDGES // NW
G = 80
NGRP = E_PER_TILE // G
N_PAD = 10240
ROWS_PER_SUB = N_PAD // NS


def _dense_body(x_ref, anc_ref, w_ref, b_ref, a_ref, h_ref, anc_out_ref):
    w = w_ref[...]
    h_ref[...] = jnp.dot(x_ref[...], w, preferred_element_type=jnp.float32)
    v = jnp.dot(anc_ref[...], w, preferred_element_type=jnp.float32) + b_ref[...]
    a = a_ref[0, 0]
    anc_out_ref[...] = jnp.maximum(v, 0.0) + a * jnp.minimum(v, 0.0)


_dense_call = pl.pallas_call(
    _dense_body,
    out_shape=(
        jax.ShapeDtypeStruct((N_NODES, D_OUT), jnp.float32),
        jax.ShapeDtypeStruct((N_GRAPHS, D_OUT), jnp.float32),
    ),
)


def _edge_body(h_hbm, src_hbm, dst_hbm, ew_hbm, out_hbm,
               src_v, dst_v, ew_v, rows_v, accum, sem):
    c = lax.axis_index("c")
    s = lax.axis_index("s")

    zero16 = jnp.zeros((16,), jnp.float32)

    @pl.loop(0, G)
    def _(i):
        for j in range(D_OUT // 16):
            rows_v[i, pl.ds(j * 16, 16)] = zero16

    @pl.loop(0, ROWS_PER_SUB // G)
    def _(k):
        pltpu.sync_copy(rows_v, accum.at[pl.ds(s * ROWS_PER_SUB + k * G, G)])

    plsc.subcore_barrier()

    base = (c * NS + s) * E_PER_TILE

    @pl.loop(0, NGRP)
    def _(g):
        off = base + g * G
        pltpu.sync_copy(src_hbm.at[pl.ds(off, G)], src_v)
        pltpu.sync_copy(ew_hbm.at[pl.ds(off, G)], ew_v)
        pltpu.sync_copy(dst_hbm.at[pl.ds(off, G)], dst_v)
        pltpu.async_copy(h_hbm.at[src_v], rows_v, sem).wait()

        @pl.loop(0, G)
        def _(i):
            w = plsc.load_gather(ew_v, [jnp.full((16,), i, jnp.int32)])
            for j in range(D_OUT // 16):
                sl = (i, pl.ds(j * 16, 16))
                rows_v[sl] = rows_v[sl] * w

        pltpu.sync_copy(rows_v, accum.at[dst_v], add=True)

    plsc.subcore_barrier()
    pltpu.sync_copy(accum.at[pl.ds(s * ROWS_PER_SUB, ROWS_PER_SUB)],
                    out_hbm.at[c].at[pl.ds(s * ROWS_PER_SUB, ROWS_PER_SUB)])


@functools.cache
def _edge_kernel():
    mesh = plsc.VectorSubcoreMesh(
        core_axis_name="c", subcore_axis_name="s",
        num_cores=NC, num_subcores=NS)
    cp = pltpu.CompilerParams()
    if "needs_layout_passes" in pltpu.CompilerParams.__dataclass_fields__:
        cp = dataclasses.replace(cp, needs_layout_passes=False)
    if "use_tc_tiling_on_sc" in pltpu.CompilerParams.__dataclass_fields__:
        cp = dataclasses.replace(cp, use_tc_tiling_on_sc=False)
    return pl.kernel(
        _edge_body,
        out_type=jax.ShapeDtypeStruct((NC, N_PAD, D_OUT), jnp.float32),
        mesh=mesh,
        compiler_params=cp,
        scratch_types=[
            pltpu.VMEM((G,), jnp.int32),
            pltpu.VMEM((G,), jnp.int32),
            pltpu.VMEM((G,), jnp.float32),
            pltpu.VMEM((G, D_OUT), jnp.float32),
            pltpu.VMEM_SHARED((N_PAD, D_OUT), jnp.float32),
            pltpu.SemaphoreType.DMA,
        ],
    )


_POOL_BS = 500
_POOL_NB = N_NODES // _POOL_BS


def _post_body(part_ref, gid_ref, b_ref, a_ref, anc_ref, wb_ref, bb_ref,
               h2_ref, pool_ref, logits_ref):
    agg = part_ref[0, :N_NODES, :] + part_ref[1, :N_NODES, :]
    v = agg + b_ref[...]
    a = a_ref[0, 0]
    h2 = jnp.maximum(v, 0.0) + a * jnp.minimum(v, 0.0)
    h2_ref[...] = h2

    gid = gid_ref[...]
    giota = lax.broadcasted_iota(jnp.int32, (1, N_GRAPHS), 1)
    ones_col = jnp.ones((_POOL_BS, 1), jnp.float32)

    acc = jnp.zeros((N_GRAPHS, D_OUT + 1), jnp.float32)
    for i in range(_POOL_NB):
        lo = i * _POOL_BS
        ids = gid[lo:lo + _POOL_BS]
        oh = (ids == giota).astype(jnp.float32)
        m = jnp.concatenate([h2[lo:lo + _POOL_BS], ones_col], axis=1)
        acc = acc + lax.dot_general(oh, m, (((0,), (0,)), ((), ())),
                                    preferred_element_type=jnp.float32)
    pool = acc[:, :D_OUT] / jnp.maximum(acc[:, D_OUT:], 1.0)
    pool_ref[...] = pool
    t = jnp.dot(anc_ref[...], wb_ref[...], preferred_element_type=jnp.float32)
    logits_ref[...] = jnp.sum(t * pool, axis=1, keepdims=True) + bb_ref[...]


_post_call = pl.pallas_call(
    _post_body,
    out_shape=(
        jax.ShapeDtypeStruct((N_NODES, D_OUT), jnp.float32),
        jax.ShapeDtypeStruct((N_GRAPHS, D_OUT), jnp.float32),
        jax.ShapeDtypeStruct((N_GRAPHS, 1), jnp.float32),
    ),
)


def kernel(x, edge_index, edge_weight, graph_ids, anchor_embs, W, b, prelu_a, Wb, bb):
    src = edge_index[0].astype(jnp.int32)
    dst = edge_index[1].astype(jnp.int32)
    gid = graph_ids.astype(jnp.int32).reshape(N_NODES, 1)
    b2 = b.reshape(1, D_OUT)
    a2 = prelu_a.reshape(1, 1)
    bb2 = bb.reshape(1, 1)

    h, anchor_out = _dense_call(x, anchor_embs, W, b2, a2)
    part = _edge_kernel()(h, src, dst, edge_weight)
    h2, pool, logits = _post_call(part, gid, b2, a2, anchor_out, Wb[0], bb2)
    return (h2, pool, anchor_out, logits)


# speedup vs baseline: 10.2483x; 2.5678x over previous
import dataclasses
import functools

import jax
import jax.numpy as jnp
from jax import lax
from jax.experimental import pallas as pl
from jax.experimental.pallas import tpu as pltpu
from jax.experimental.pallas import tpu_sc as plsc

N_NODES = 10000
N_EDGES = 320000
D_IN = 128
D_OUT = 64
N_GRAPHS = 2500

NC = 2
NS = 16
NW = NC * NS
E_PER_TILE = N_EDGES // NW
G = 80
NGRP = E_PER_TILE // G
N_PAD = 10240
ROWS_PER_SUB = N_PAD // NS


def _dense_body(x_ref, anc_ref, w_ref, b_ref, a_ref, h_ref, anc_out_ref):
    w = w_ref[...]
    h_ref[...] = jnp.dot(x_ref[...], w, preferred_element_type=jnp.float32)
    v = jnp.dot(anc_ref[...], w, preferred_element_type=jnp.float32) + b_ref[...]
    a = a_ref[0, 0]
    anc_out_ref[...] = jnp.maximum(v, 0.0) + a * jnp.minimum(v, 0.0)


_dense_call = pl.pallas_call(
    _dense_body,
    out_shape=(
        jax.ShapeDtypeStruct((N_NODES, D_OUT), jnp.float32),
        jax.ShapeDtypeStruct((N_GRAPHS, D_OUT), jnp.float32),
    ),
)


NBUF = 4


def _edge_body(h_hbm, src_hbm, dst_hbm, ew_hbm, out_hbm,
               src_v, dst_v, ew_v, rows_a, rows_b, accum,
               isem, gsems, ssems):
    c = lax.axis_index("c")
    s = lax.axis_index("s")
    w = c * NS + s

    cp1 = pltpu.async_copy(src_hbm.at[pl.ds(w * NGRP, NGRP)], src_v, isem)
    cp2 = pltpu.async_copy(dst_hbm.at[pl.ds(w * NGRP, NGRP)], dst_v, isem)
    cp3 = pltpu.async_copy(ew_hbm.at[pl.ds(w * E_PER_TILE, E_PER_TILE)],
                           ew_v, isem)

    zero16 = jnp.zeros((16,), jnp.float32)

    @pl.loop(0, G)
    def _(i):
        for j in range(D_OUT // 16):
            rows_a[0][i, pl.ds(j * 16, 16)] = zero16

    @pl.loop(0, ROWS_PER_SUB // G)
    def _(k):
        pltpu.sync_copy(rows_a[0], accum.at[pl.ds(s * ROWS_PER_SUB + k * G, G)])

    cp1.wait()
    cp2.wait()
    cp3.wait()
    plsc.subcore_barrier()

    def scale(g, src_buf, dst_buf):
        @pl.loop(0, G)
        def _(i):
            wv = plsc.load_gather(ew_v, [jnp.full((16,), g * G + i, jnp.int32)])
            for j in range(D_OUT // 16):
                sl = (i, pl.ds(j * 16, 16))
                dst_buf[sl] = src_buf[sl] * wv

    for b in range(NBUF):
        pltpu.async_copy(h_hbm.at[src_v.at[b]], rows_a[b], gsems[b])
    for b in range(NBUF):
        pltpu.make_async_copy(h_hbm.at[src_v.at[b]], rows_a[b], gsems[b]).wait()
        scale(b, rows_a[b], rows_b[b])
        pltpu.async_copy(h_hbm.at[src_v.at[b + NBUF]], rows_a[b], gsems[b])
        pltpu.async_copy(rows_b[b], accum.at[dst_v.at[b]], ssems[b], add=True)

    STEADY_END = ((NGRP - NBUF) // NBUF) * NBUF + NBUF

    @pl.loop(NBUF, STEADY_END, step=NBUF)
    def _(g0):
        for b in range(NBUF):
            g = g0 + b
            pltpu.make_async_copy(h_hbm.at[src_v.at[g]], rows_a[b],
                                  gsems[b]).wait()
            pltpu.make_async_copy(rows_b[b], accum.at[dst_v.at[g]],
                                  ssems[b]).wait()
            scale(g, rows_a[b], rows_b[b])

            @pl.when(g + NBUF < NGRP)
            def _():
                pltpu.async_copy(h_hbm.at[src_v.at[g + NBUF]], rows_a[b],
                                 gsems[b])
            pltpu.async_copy(rows_b[b], accum.at[dst_v.at[g]], ssems[b],
                             add=True)

    for g in range(STEADY_END, NGRP):
        b = g % NBUF
        pltpu.make_async_copy(h_hbm.at[src_v.at[g]], rows_a[b], gsems[b]).wait()
        pltpu.make_async_copy(rows_b[b], accum.at[dst_v.at[g]], ssems[b]).wait()
        scale(g, rows_a[b], rows_b[b])
        pltpu.async_copy(rows_b[b], accum.at[dst_v.at[g]], ssems[b], add=True)

    for b in range(NBUF):
        pltpu.make_async_copy(rows_b[b], accum.at[dst_v.at[0]], ssems[b]).wait()

    plsc.subcore_barrier()
    pltpu.sync_copy(accum.at[pl.ds(s * ROWS_PER_SUB, ROWS_PER_SUB)],
                    out_hbm.at[c].at[pl.ds(s * ROWS_PER_SUB, ROWS_PER_SUB)])


@functools.cache
def _edge_kernel():
    mesh = plsc.VectorSubcoreMesh(
        core_axis_name="c", subcore_axis_name="s",
        num_cores=NC, num_subcores=NS)
    cp = pltpu.CompilerParams()
    if "needs_layout_passes" in pltpu.CompilerParams.__dataclass_fields__:
        cp = dataclasses.replace(cp, needs_layout_passes=False)
    if "use_tc_tiling_on_sc" in pltpu.CompilerParams.__dataclass_fields__:
        cp = dataclasses.replace(cp, use_tc_tiling_on_sc=False)
    return pl.kernel(
        _edge_body,
        out_type=jax.ShapeDtypeStruct((NC, N_PAD, D_OUT), jnp.float32),
        mesh=mesh,
        compiler_params=cp,
        scratch_types=[
            pltpu.VMEM((NGRP, G), jnp.int32),
            pltpu.VMEM((NGRP, G), jnp.int32),
            pltpu.VMEM((E_PER_TILE,), jnp.float32),
            [pltpu.VMEM((G, D_OUT), jnp.float32) for _ in range(NBUF)],
            [pltpu.VMEM((G, D_OUT), jnp.float32) for _ in range(NBUF)],
            pltpu.VMEM_SHARED((N_PAD, D_OUT), jnp.float32),
            pltpu.SemaphoreType.DMA,
            [pltpu.SemaphoreType.DMA for _ in range(NBUF)],
            [pltpu.SemaphoreType.DMA for _ in range(NBUF)],
        ],
    )


_POOL_BS = 500
_POOL_NB = N_NODES // _POOL_BS


def _post_body(part_ref, gid_ref, b_ref, a_ref, anc_ref, wb_ref, bb_ref,
               h2_ref, pool_ref, logits_ref):
    agg = part_ref[0, :N_NODES, :] + part_ref[1, :N_NODES, :]
    v = agg + b_ref[...]
    a = a_ref[0, 0]
    h2 = jnp.maximum(v, 0.0) + a * jnp.minimum(v, 0.0)
    h2_ref[...] = h2

    gid = gid_ref[...]
    giota = lax.broadcasted_iota(jnp.int32, (1, N_GRAPHS), 1)
    ones_col = jnp.ones((_POOL_BS, 1), jnp.float32)

    acc = jnp.zeros((N_GRAPHS, D_OUT + 1), jnp.float32)
    for i in range(_POOL_NB):
        lo = i * _POOL_BS
        ids = gid[lo:lo + _POOL_BS]
        oh = (ids == giota).astype(jnp.float32)
        m = jnp.concatenate([h2[lo:lo + _POOL_BS], ones_col], axis=1)
        acc = acc + lax.dot_general(oh, m, (((0,), (0,)), ((), ())),
                                    preferred_element_type=jnp.float32)
    pool = acc[:, :D_OUT] / jnp.maximum(acc[:, D_OUT:], 1.0)
    pool_ref[...] = pool
    t = jnp.dot(anc_ref[...], wb_ref[...], preferred_element_type=jnp.float32)
    logits_ref[...] = jnp.sum(t * pool, axis=1, keepdims=True) + bb_ref[...]


_post_call = pl.pallas_call(
    _post_body,
    out_shape=(
        jax.ShapeDtypeStruct((N_NODES, D_OUT), jnp.float32),
        jax.ShapeDtypeStruct((N_GRAPHS, D_OUT), jnp.float32),
        jax.ShapeDtypeStruct((N_GRAPHS, 1), jnp.float32),
    ),
)


def kernel(x, edge_index, edge_weight, graph_ids, anchor_embs, W, b, prelu_a, Wb, bb):
    src = edge_index[0].astype(jnp.int32).reshape(NW * NGRP, G)
    dst = edge_index[1].astype(jnp.int32).reshape(NW * NGRP, G)
    gid = graph_ids.astype(jnp.int32).reshape(N_NODES, 1)
    b2 = b.reshape(1, D_OUT)
    a2 = prelu_a.reshape(1, 1)
    bb2 = bb.reshape(1, 1)

    h, anchor_out = _dense_call(x, anchor_embs, W, b2, a2)
    part = _edge_kernel()(h, src, dst, edge_weight)
    h2, pool, logits = _post_call(part, gid, b2, a2, anchor_out, Wb[0], bb2)
    return (h2, pool, anchor_out, logits)


# speedup vs baseline: 12.8428x; 1.2532x over previous
import dataclasses
import functools

import jax
import jax.numpy as jnp
from jax import lax
from jax.experimental import pallas as pl
from jax.experimental.pallas import tpu as pltpu
from jax.experimental.pallas import tpu_sc as plsc

N_NODES = 10000
N_EDGES = 320000
D_IN = 128
D_OUT = 64
N_GRAPHS = 2500

NC = 2
NS = 16
NW = NC * NS
E_PER_TILE = N_EDGES // NW
G = 80
NGRP = E_PER_TILE // G
N_PAD = 10240
ROWS_PER_SUB = N_PAD // NS


def _dense_body(x_ref, anc_ref, w_ref, b_ref, a_ref, h_ref, anc_out_ref):
    w = w_ref[...]
    h_ref[...] = jnp.dot(x_ref[...], w, preferred_element_type=jnp.float32)
    v = jnp.dot(anc_ref[...], w, preferred_element_type=jnp.float32) + b_ref[...]
    a = a_ref[0, 0]
    anc_out_ref[...] = jnp.maximum(v, 0.0) + a * jnp.minimum(v, 0.0)


_dense_call = pl.pallas_call(
    _dense_body,
    out_shape=(
        jax.ShapeDtypeStruct((N_NODES, D_OUT), jnp.float32),
        jax.ShapeDtypeStruct((N_GRAPHS, D_OUT), jnp.float32),
    ),
)


NBUF = 4


def _edge_body(h_hbm, eidx_hbm, ew_hbm, out_hbm,
               src_v, dst_v, ew_v, rows_a, rows_b, accum,
               isem, gsems, ssems):
    c = lax.axis_index("c")
    s = lax.axis_index("s")
    w = c * NS + s

    cp1 = pltpu.async_copy(eidx_hbm.at[0, pl.ds(w * NGRP, NGRP)], src_v, isem)
    cp2 = pltpu.async_copy(eidx_hbm.at[1, pl.ds(w * NGRP, NGRP)], dst_v, isem)
    cp3 = pltpu.async_copy(ew_hbm.at[pl.ds(w * E_PER_TILE, E_PER_TILE)],
                           ew_v, isem)

    zero16 = jnp.zeros((16,), jnp.float32)

    @pl.loop(0, G)
    def _(i):
        for j in range(D_OUT // 16):
            rows_a[0][i, pl.ds(j * 16, 16)] = zero16

    @pl.loop(0, ROWS_PER_SUB // G)
    def _(k):
        pltpu.sync_copy(rows_a[0], accum.at[pl.ds(s * ROWS_PER_SUB + k * G, G)])

    cp1.wait()
    cp2.wait()
    cp3.wait()
    plsc.subcore_barrier()

    def scale(g, src_buf, dst_buf):
        @plsc.parallel_loop(0, G, unroll=4)
        def _(i):
            wv = plsc.load_gather(ew_v, [jnp.full((16,), g * G + i, jnp.int32)])
            for j in range(D_OUT // 16):
                sl = (i, pl.ds(j * 16, 16))
                dst_buf[sl] = src_buf[sl] * wv

    for b in range(NBUF):
        pltpu.async_copy(h_hbm.at[src_v.at[b]], rows_a[b], gsems[b])
    for b in range(NBUF):
        pltpu.make_async_copy(h_hbm.at[src_v.at[b]], rows_a[b], gsems[b]).wait()
        scale(b, rows_a[b], rows_b[b])
        pltpu.async_copy(h_hbm.at[src_v.at[b + NBUF]], rows_a[b], gsems[b])
        pltpu.async_copy(rows_b[b], accum.at[dst_v.at[b]], ssems[b], add=True)

    STEADY_END = ((NGRP - NBUF) // NBUF) * NBUF + NBUF

    @pl.loop(NBUF, STEADY_END, step=NBUF)
    def _(g0):
        for b in range(NBUF):
            g = g0 + b
            pltpu.make_async_copy(h_hbm.at[src_v.at[g]], rows_a[b],
                                  gsems[b]).wait()
            pltpu.make_async_copy(rows_b[b], accum.at[dst_v.at[g]],
                                  ssems[b]).wait()
            scale(g, rows_a[b], rows_b[b])

            @pl.when(g + NBUF < NGRP)
            def _():
                pltpu.async_copy(h_hbm.at[src_v.at[g + NBUF]], rows_a[b],
                                 gsems[b])
            pltpu.async_copy(rows_b[b], accum.at[dst_v.at[g]], ssems[b],
                             add=True)

    for g in range(STEADY_END, NGRP):
        b = g % NBUF
        pltpu.make_async_copy(h_hbm.at[src_v.at[g]], rows_a[b], gsems[b]).wait()
        pltpu.make_async_copy(rows_b[b], accum.at[dst_v.at[g]], ssems[b]).wait()
        scale(g, rows_a[b], rows_b[b])
        pltpu.async_copy(rows_b[b], accum.at[dst_v.at[g]], ssems[b], add=True)

    for b in range(NBUF):
        pltpu.make_async_copy(rows_b[b], accum.at[dst_v.at[0]], ssems[b]).wait()

    plsc.subcore_barrier()
    pltpu.sync_copy(accum.at[pl.ds(s * ROWS_PER_SUB, ROWS_PER_SUB)],
                    out_hbm.at[c].at[pl.ds(s * ROWS_PER_SUB, ROWS_PER_SUB)])


@functools.cache
def _edge_kernel():
    mesh = plsc.VectorSubcoreMesh(
        core_axis_name="c", subcore_axis_name="s",
        num_cores=NC, num_subcores=NS)
    cp = pltpu.CompilerParams()
    if "needs_layout_passes" in pltpu.CompilerParams.__dataclass_fields__:
        cp = dataclasses.replace(cp, needs_layout_passes=False)
    if "use_tc_tiling_on_sc" in pltpu.CompilerParams.__dataclass_fields__:
        cp = dataclasses.replace(cp, use_tc_tiling_on_sc=False)
    return pl.kernel(
        _edge_body,
        out_type=jax.ShapeDtypeStruct((NC, N_PAD, D_OUT), jnp.float32),
        mesh=mesh,
        compiler_params=cp,
        scratch_types=[
            pltpu.VMEM((NGRP, G), jnp.int32),
            pltpu.VMEM((NGRP, G), jnp.int32),
            pltpu.VMEM((E_PER_TILE,), jnp.float32),
            [pltpu.VMEM((G, D_OUT), jnp.float32) for _ in range(NBUF)],
            [pltpu.VMEM((G, D_OUT), jnp.float32) for _ in range(NBUF)],
            pltpu.VMEM_SHARED((N_PAD, D_OUT), jnp.float32),
            pltpu.SemaphoreType.DMA,
            [pltpu.SemaphoreType.DMA for _ in range(NBUF)],
            [pltpu.SemaphoreType.DMA for _ in range(NBUF)],
        ],
    )


_POOL_BS = 500
_POOL_NB = N_NODES // _POOL_BS


def _post_body(part_ref, gid_ref, b_ref, a_ref, anc_ref, wb_ref, bb_ref,
               h2_ref, pool_ref, logits_ref):
    agg = part_ref[0, :N_NODES, :] + part_ref[1, :N_NODES, :]
    v = agg + b_ref[...]
    a = a_ref[0, 0]
    h2 = jnp.maximum(v, 0.0) + a * jnp.minimum(v, 0.0)
    h2_ref[...] = h2

    gid = gid_ref[...]
    giota = lax.broadcasted_iota(jnp.int32, (1, N_GRAPHS), 1)
    ones_col = jnp.ones((_POOL_BS, 1), jnp.float32)

    acc = jnp.zeros((N_GRAPHS, D_OUT + 1), jnp.float32)
    for i in range(_POOL_NB):
        lo = i * _POOL_BS
        ids = gid[lo:lo + _POOL_BS]
        oh = (ids == giota).astype(jnp.float32)
        m = jnp.concatenate([h2[lo:lo + _POOL_BS], ones_col], axis=1)
        acc = acc + lax.dot_general(oh, m, (((0,), (0,)), ((), ())),
                                    preferred_element_type=jnp.float32)
    pool = acc[:, :D_OUT] / jnp.maximum(acc[:, D_OUT:], 1.0)
    pool_ref[...] = pool
    t = jnp.dot(anc_ref[...], wb_ref[...], preferred_element_type=jnp.float32)
    logits_ref[...] = jnp.sum(t * pool, axis=1, keepdims=True) + bb_ref[...]


_post_call = pl.pallas_call(
    _post_body,
    out_shape=(
        jax.ShapeDtypeStruct((N_NODES, D_OUT), jnp.float32),
        jax.ShapeDtypeStruct((N_GRAPHS, D_OUT), jnp.float32),
        jax.ShapeDtypeStruct((N_GRAPHS, 1), jnp.float32),
    ),
)


def kernel(x, edge_index, edge_weight, graph_ids, anchor_embs, W, b, prelu_a, Wb, bb):
    eidx = edge_index.astype(jnp.int32).reshape(2, NW * NGRP, G)
    gid = graph_ids.astype(jnp.int32).reshape(N_NODES, 1)
    b2 = b.reshape(1, D_OUT)
    a2 = prelu_a.reshape(1, 1)
    bb2 = bb.reshape(1, 1)

    h, anchor_out = _dense_call(x, anchor_embs, W, b2, a2)
    part = _edge_kernel()(h, eidx, edge_weight)
    h2, pool, logits = _post_call(part, gid, b2, a2, anchor_out, Wb[0], bb2)
    return (h2, pool, anchor_out, logits)


# speedup vs baseline: 14.7009x; 1.1447x over previous
import dataclasses
import functools

import jax
import jax.numpy as jnp
from jax import lax
from jax.experimental import pallas as pl
from jax.experimental.pallas import tpu as pltpu
from jax.experimental.pallas import tpu_sc as plsc

N_NODES = 10000
N_EDGES = 320000
D_IN = 128
D_OUT = 64
N_GRAPHS = 2500

NC = 2
NS = 16
NW = NC * NS
E_PER_TILE = N_EDGES // NW
G = 80
NGRP = E_PER_TILE // G
N_PAD = 10240
ROWS_PER_SUB = N_PAD // NS


def _dense_body(x_ref, anc_ref, w_ref, b_ref, a_ref, h_ref, anc_out_ref):
    w = w_ref[...]
    h_ref[...] = jnp.dot(x_ref[...], w, preferred_element_type=jnp.float32)
    v = jnp.dot(anc_ref[...], w, preferred_element_type=jnp.float32) + b_ref[...]
    a = a_ref[0, 0]
    anc_out_ref[...] = jnp.maximum(v, 0.0) + a * jnp.minimum(v, 0.0)


_dense_call = pl.pallas_call(
    _dense_body,
    out_shape=(
        jax.ShapeDtypeStruct((N_NODES, D_OUT), jnp.float32),
        jax.ShapeDtypeStruct((N_GRAPHS, D_OUT), jnp.float32),
    ),
)


NBUF = 4


def _edge_body(h_hbm, eidx_hbm, ew_hbm, out_hbm,
               src_v, dst_v, ew_v, rows_a, rows_b, accum,
               isem, gsems, ssems):
    c = lax.axis_index("c")
    s = lax.axis_index("s")
    w = c * NS + s

    cp1 = pltpu.async_copy(eidx_hbm.at[0, pl.ds(w * NGRP, NGRP)], src_v, isem)
    cp2 = pltpu.async_copy(eidx_hbm.at[1, pl.ds(w * NGRP, NGRP)], dst_v, isem)
    cp3 = pltpu.async_copy(ew_hbm.at[pl.ds(w * E_PER_TILE, E_PER_TILE)],
                           ew_v, isem)

    zero16 = jnp.zeros((16,), jnp.float32)

    @pl.loop(0, G)
    def _(i):
        for j in range(D_OUT // 16):
            rows_a[0][i, pl.ds(j * 16, 16)] = zero16

    @pl.loop(0, ROWS_PER_SUB // G)
    def _(k):
        pltpu.sync_copy(rows_a[0], accum.at[pl.ds(s * ROWS_PER_SUB + k * G, G)])

    cp1.wait()
    cp2.wait()
    cp3.wait()
    plsc.subcore_barrier()

    def scale(g, src_buf, dst_buf):
        @plsc.parallel_loop(0, G, unroll=4)
        def _(i):
            wv = plsc.load_gather(ew_v, [jnp.full((16,), g * G + i, jnp.int32)])
            for j in range(D_OUT // 16):
                sl = (i, pl.ds(j * 16, 16))
                dst_buf[sl] = src_buf[sl] * wv

    for b in range(NBUF):
        pltpu.async_copy(h_hbm.at[src_v.at[b]], rows_a[b], gsems[b])
    for b in range(NBUF):
        pltpu.make_async_copy(h_hbm.at[src_v.at[b]], rows_a[b], gsems[b]).wait()
        scale(b, rows_a[b], rows_b[b])
        pltpu.async_copy(h_hbm.at[src_v.at[b + NBUF]], rows_a[b], gsems[b])
        pltpu.async_copy(rows_b[b], accum.at[dst_v.at[b]], ssems[b], add=True)

    STEADY_END = ((NGRP - NBUF) // NBUF) * NBUF + NBUF

    @pl.loop(NBUF, STEADY_END, step=NBUF)
    def _(g0):
        for b in range(NBUF):
            g = g0 + b
            pltpu.make_async_copy(h_hbm.at[src_v.at[g]], rows_a[b],
                                  gsems[b]).wait()
            pltpu.make_async_copy(rows_b[b], accum.at[dst_v.at[g]],
                                  ssems[b]).wait()
            scale(g, rows_a[b], rows_b[b])

            @pl.when(g + NBUF < NGRP)
            def _():
                pltpu.async_copy(h_hbm.at[src_v.at[g + NBUF]], rows_a[b],
                                 gsems[b])
            pltpu.async_copy(rows_b[b], accum.at[dst_v.at[g]], ssems[b],
                             add=True)

    for g in range(STEADY_END, NGRP):
        b = g % NBUF
        pltpu.make_async_copy(h_hbm.at[src_v.at[g]], rows_a[b], gsems[b]).wait()
        pltpu.make_async_copy(rows_b[b], accum.at[dst_v.at[g]], ssems[b]).wait()
        scale(g, rows_a[b], rows_b[b])
        pltpu.async_copy(rows_b[b], accum.at[dst_v.at[g]], ssems[b], add=True)

    for b in range(NBUF):
        pltpu.make_async_copy(rows_b[b], accum.at[dst_v.at[0]], ssems[b]).wait()

    plsc.subcore_barrier()
    pltpu.sync_copy(accum.at[pl.ds(s * ROWS_PER_SUB, ROWS_PER_SUB)],
                    out_hbm.at[c].at[pl.ds(s * ROWS_PER_SUB, ROWS_PER_SUB)])


@functools.cache
def _edge_kernel():
    mesh = plsc.VectorSubcoreMesh(
        core_axis_name="c", subcore_axis_name="s",
        num_cores=NC, num_subcores=NS)
    cp = pltpu.CompilerParams()
    if "needs_layout_passes" in pltpu.CompilerParams.__dataclass_fields__:
        cp = dataclasses.replace(cp, needs_layout_passes=False)
    if "use_tc_tiling_on_sc" in pltpu.CompilerParams.__dataclass_fields__:
        cp = dataclasses.replace(cp, use_tc_tiling_on_sc=False)
    return pl.kernel(
        _edge_body,
        out_type=jax.ShapeDtypeStruct((NC, N_PAD, D_OUT), jnp.float32),
        mesh=mesh,
        compiler_params=cp,
        scratch_types=[
            pltpu.VMEM((NGRP, G), jnp.int32),
            pltpu.VMEM((NGRP, G), jnp.int32),
            pltpu.VMEM((E_PER_TILE,), jnp.float32),
            [pltpu.VMEM((G, D_OUT), jnp.float32) for _ in range(NBUF)],
            [pltpu.VMEM((G, D_OUT), jnp.float32) for _ in range(NBUF)],
            pltpu.VMEM_SHARED((N_PAD, D_OUT), jnp.float32),
            pltpu.SemaphoreType.DMA,
            [pltpu.SemaphoreType.DMA for _ in range(NBUF)],
            [pltpu.SemaphoreType.DMA for _ in range(NBUF)],
        ],
    )


POOL_PAD = 2560
CROWS = POOL_PAD // NS
NCHUNK = N_NODES // G
KSLOT = 4


def _pool_body(part_hbm, gid_hbm, bvec_hbm, avec_hbm,
               h2_hbm, pool_hbm, cnt_hbm,
               p0_v, p1_v, h2_v, gid_v, ones_v, zb64, zb16, bvec_v, avec_v,
               pacc, cacc, isem, lsems, hsems, psems, csems):
    c = lax.axis_index("c")
    s = lax.axis_index("s")
    t = c * NS + s

    cpb = pltpu.async_copy(bvec_hbm, bvec_v, isem)
    cpa = pltpu.async_copy(avec_hbm, avec_v, isem)

    ones16 = jnp.ones((16,), jnp.float32)
    zero16 = jnp.zeros((16,), jnp.float32)

    @pl.loop(0, G)
    def _(i):
        ones_v[i, pl.ds(0, 16)] = ones16
        zb16[i, pl.ds(0, 16)] = zero16
        for j in range(D_OUT // 16):
            zb64[i, pl.ds(j * 16, 16)] = zero16

    for k in range(CROWS // G):
        pltpu.sync_copy(zb64, pacc.at[pl.ds(s * CROWS + k * G, G)])
        pltpu.sync_copy(zb16, cacc.at[pl.ds(s * CROWS + k * G, G)])

    cpb.wait()
    cpa.wait()
    plsc.subcore_barrier()

    av = avec_v[pl.ds(0, 16)]
    bvs = [bvec_v[pl.ds(j * 16, 16)] for j in range(D_OUT // 16)]

    for k in range(KSLOT):
        chunk = t + k * NW

        @pl.when(chunk < NCHUNK)
        def _():
            off = chunk * G
            pltpu.async_copy(part_hbm.at[0, pl.ds(off, G)], p0_v[k], lsems[k])
            pltpu.async_copy(part_hbm.at[1, pl.ds(off, G)], p1_v[k], lsems[k])
            pltpu.async_copy(gid_hbm.at[pl.ds(off, G)], gid_v[k], lsems[k])

    for k in range(KSLOT):
        chunk = t + k * NW

        @pl.when(chunk < NCHUNK)
        def _():
            off = chunk * G
            pltpu.make_async_copy(part_hbm.at[0, pl.ds(off, G)], p0_v[k],
                                  lsems[k]).wait()
            pltpu.make_async_copy(part_hbm.at[1, pl.ds(off, G)], p1_v[k],
                                  lsems[k]).wait()
            pltpu.make_async_copy(gid_hbm.at[pl.ds(off, G)], gid_v[k],
                                  lsems[k]).wait()

            @plsc.parallel_loop(0, G, unroll=2)
            def _(i):
                for j in range(D_OUT // 16):
                    sl = (i, pl.ds(j * 16, 16))
                    v = p0_v[k][sl] + p1_v[k][sl] + bvs[j]
                    h2_v[k][sl] = (jnp.maximum(v, 0.0)
                                   + av * jnp.minimum(v, 0.0))

            pltpu.async_copy(h2_v[k], h2_hbm.at[pl.ds(off, G)], hsems[k])
            pltpu.async_copy(h2_v[k], pacc.at[gid_v[k]], psems[k], add=True)
            pltpu.async_copy(ones_v, cacc.at[gid_v[k]], csems[k], add=True)

    for k in range(KSLOT):
        chunk = t + k * NW

        @pl.when(chunk < NCHUNK)
        def _():
            off = chunk * G
            pltpu.make_async_copy(h2_v[k], h2_hbm.at[pl.ds(off, G)],
                                  hsems[k]).wait()
            pltpu.make_async_copy(h2_v[k], pacc.at[gid_v[k]], psems[k]).wait()
            pltpu.make_async_copy(ones_v, cacc.at[gid_v[k]], csems[k]).wait()

    plsc.subcore_barrier()
    pltpu.sync_copy(pacc.at[pl.ds(s * CROWS, CROWS)],
                    pool_hbm.at[c].at[pl.ds(s * CROWS, CROWS)])
    pltpu.sync_copy(cacc.at[pl.ds(s * CROWS, CROWS)],
                    cnt_hbm.at[c].at[pl.ds(s * CROWS, CROWS)])


@functools.cache
def _pool_kernel():
    mesh = plsc.VectorSubcoreMesh(
        core_axis_name="c", subcore_axis_name="s",
        num_cores=NC, num_subcores=NS)
    cp = pltpu.CompilerParams()
    if "needs_layout_passes" in pltpu.CompilerParams.__dataclass_fields__:
        cp = dataclasses.replace(cp, needs_layout_passes=False)
    if "use_tc_tiling_on_sc" in pltpu.CompilerParams.__dataclass_fields__:
        cp = dataclasses.replace(cp, use_tc_tiling_on_sc=False)
    return pl.kernel(
        _pool_body,
        out_type=(
            jax.ShapeDtypeStruct((N_NODES, D_OUT), jnp.float32),
            jax.ShapeDtypeStruct((NC, POOL_PAD, D_OUT), jnp.float32),
            jax.ShapeDtypeStruct((NC, POOL_PAD, 16), jnp.float32),
        ),
        mesh=mesh,
        compiler_params=cp,
        scratch_types=[
            [pltpu.VMEM((G, D_OUT), jnp.float32) for _ in range(KSLOT)],
            [pltpu.VMEM((G, D_OUT), jnp.float32) for _ in range(KSLOT)],
            [pltpu.VMEM((G, D_OUT), jnp.float32) for _ in range(KSLOT)],
            [pltpu.VMEM((G,), jnp.int32) for _ in range(KSLOT)],
            pltpu.VMEM((G, 16), jnp.float32),
            pltpu.VMEM((G, D_OUT), jnp.float32),
            pltpu.VMEM((G, 16), jnp.float32),
            pltpu.VMEM((D_OUT,), jnp.float32),
            pltpu.VMEM((16,), jnp.float32),
            pltpu.VMEM_SHARED((POOL_PAD, D_OUT), jnp.float32),
            pltpu.VMEM_SHARED((POOL_PAD, 16), jnp.float32),
            pltpu.SemaphoreType.DMA,
            [pltpu.SemaphoreType.DMA for _ in range(KSLOT)],
            [pltpu.SemaphoreType.DMA for _ in range(KSLOT)],
            [pltpu.SemaphoreType.DMA for _ in range(KSLOT)],
            [pltpu.SemaphoreType.DMA for _ in range(KSLOT)],
        ],
    )


def _final_body(pp_ref, cc_ref, anc_ref, wb_ref, bb_ref,
                pool_ref, logits_ref):
    seg = pp_ref[0, :N_GRAPHS, :] + pp_ref[1, :N_GRAPHS, :]
    cnt = cc_ref[0, :N_GRAPHS, 0:1] + cc_ref[1, :N_GRAPHS, 0:1]
    pool = seg / jnp.maximum(cnt, 1.0)
    pool_ref[...] = pool
    t2 = jnp.dot(anc_ref[...], wb_ref[...], preferred_element_type=jnp.float32)
    logits_ref[...] = jnp.sum(t2 * pool, axis=1, keepdims=True) + bb_ref[...]


_final_call = pl.pallas_call(
    _final_body,
    out_shape=(
        jax.ShapeDtypeStruct((N_GRAPHS, D_OUT), jnp.float32),
        jax.ShapeDtypeStruct((N_GRAPHS, 1), jnp.float32),
    ),
)


def kernel(x, edge_index, edge_weight, graph_ids, anchor_embs, W, b, prelu_a, Wb, bb):
    eidx = edge_index.astype(jnp.int32).reshape(2, NW * NGRP, G)
    gid = graph_ids.astype(jnp.int32)
    b2 = b.reshape(1, D_OUT)
    a2 = prelu_a.reshape(1, 1)
    bb2 = bb.reshape(1, 1)
    avec = jnp.broadcast_to(prelu_a.reshape(1), (16,))

    h, anchor_out = _dense_call(x, anchor_embs, W, b2, a2)
    part = _edge_kernel()(h, eidx, edge_weight)
    h2, pool_part, cnt_part = _pool_kernel()(part, gid, b, avec)
    pool, logits = _final_call(pool_part, cnt_part, anchor_out, Wb[0], bb2)
    return (h2, pool, anchor_out, logits)
